# baseline (device time: 26501 ns/iter reference)
import jax
import jax.numpy as jnp
from jax import lax
from jax.experimental import pallas as pl
from jax.experimental.pallas import tpu as pltpu

BM = 512


def _partial_body(x_ref, dy_ref, out_ref):
    i = pl.program_id(0)
    x = x_ref[...]
    dy = dy_ref[...]
    d = x.shape[1]
    mu = jnp.sum(x, axis=1, keepdims=True) * (1.0 / d)
    xc = x - mu
    var = jnp.sum(xc * xc, axis=1, keepdims=True) * (1.0 / d)
    rstd = lax.rsqrt(var + 1e-5)
    xhat = xc * rstd
    dgamma = jnp.sum(dy * xhat, axis=0, keepdims=True)
    dbeta = jnp.sum(dy, axis=0, keepdims=True)
    part = jnp.concatenate([dgamma, dbeta], axis=0)

    @pl.when(i == 0)
    def _():
        out_ref[...] = part

    @pl.when(i != 0)
    def _():
        out_ref[...] += part


def _allreduce_body(p_ref, out_ref, comm_ref, send_sem, recv_sem):
    my_x = lax.axis_index("x")
    my_y = lax.axis_index("y")
    peer = (1 - my_x, my_y)

    barrier = pltpu.get_barrier_semaphore()
    pl.semaphore_signal(
        barrier, inc=1, device_id=peer, device_id_type=pl.DeviceIdType.MESH
    )
    pl.semaphore_wait(barrier, 1)

    rdma = pltpu.make_async_remote_copy(
        src_ref=p_ref,
        dst_ref=comm_ref,
        send_sem=send_sem,
        recv_sem=recv_sem,
        device_id=peer,
        device_id_type=pl.DeviceIdType.MESH,
    )
    rdma.start()
    rdma.wait()

    out_ref[...] = p_ref[...] + comm_ref[...]


def kernel(x, dy, gamma):
    del gamma
    m, d = x.shape

    partial = pl.pallas_call(
        _partial_body,
        grid=(m // BM,),
        in_specs=[
            pl.BlockSpec((BM, d), lambda i: (i, 0)),
            pl.BlockSpec((BM, d), lambda i: (i, 0)),
        ],
        out_specs=pl.BlockSpec((2, d), lambda i: (0, 0)),
        out_shape=jax.ShapeDtypeStruct((2, d), jnp.float32),
    )(x, dy)

    return pl.pallas_call(
        _allreduce_body,
        out_shape=jax.ShapeDtypeStruct((2, d), jnp.float32),
        in_specs=[pl.BlockSpec(memory_space=pltpu.VMEM)],
        out_specs=pl.BlockSpec(memory_space=pltpu.VMEM),
        scratch_shapes=[
            pltpu.VMEM((2, d), jnp.float32),
            pltpu.SemaphoreType.DMA,
            pltpu.SemaphoreType.DMA,
        ],
        compiler_params=pltpu.CompilerParams(collective_id=0),
    )(partial)
